# baseline (speedup 1.0000x reference)
import jax
import jax.numpy as jnp
from jax.experimental import pallas as pl
from jax.experimental.pallas import tpu as pltpu

_E = 16
_NF = 1
_NFS = 2


def _routing_body(x, wg):
    T = x.shape[0]
    E = wg.shape[1]
    logits = jax.lax.dot_general(
        wg, x, (((0,), (1,)), ((), ())),
        preferred_element_type=jnp.float32)
    ids = jax.lax.broadcasted_iota(jnp.int32, (E, T), 0)
    m1 = jnp.max(logits, axis=0, keepdims=True)
    a = jnp.min(jnp.where(logits == m1, ids, E), axis=0, keepdims=True)
    masked = jnp.where(ids == a, -jnp.inf, logits)
    m2 = jnp.max(masked, axis=0, keepdims=True)
    b = jnp.min(jnp.where(masked == m2, ids, E), axis=0, keepdims=True)
    e2 = jnp.exp(m2 - m1)
    p1 = 1.0 / (1.0 + e2)
    p2 = e2 / (1.0 + e2)
    g0 = jnp.where(a == 0, p1, 0.0) + jnp.where(b == 0, p2, 0.0)
    g1 = jnp.where(a == 1, p1, 0.0) + jnp.where(b == 1, p2, 0.0)
    coeff = jnp.where(ids == a, g0, 0.0) + jnp.where(ids == b, g1, 0.0)

    act_col = (jnp.max(coeff, axis=1, keepdims=True) > 0.0).astype(jnp.float32)
    r = jax.lax.broadcasted_iota(jnp.int32, (E, E), 0)
    c = jax.lax.broadcasted_iota(jnp.int32, (E, E), 1)
    eye = (r == c).astype(jnp.float32)
    lstrict = (c < r).astype(jnp.float32)
    pos_col = jax.lax.dot_general(
        lstrict, act_col, (((1,), (0,)), ((), ())),
        preferred_element_type=jnp.float32)

    def _t(col):
        return jax.lax.dot_general(col, eye, (((0,), (0,)), ((), ())),
                                   preferred_element_type=jnp.float32)

    act_row = _t(act_col)
    pos_row = _t(pos_col)
    na = jnp.sum(act_row)
    npad = E - na
    rf = r.astype(jnp.float32)
    first = jnp.where((act_row > 0.0) & (pos_row + npad == rf), 1.0, 0.0)
    padm = jnp.where((act_row > 0.0) & (pos_row == 0.0) & (rf < npad),
                     1.0, 0.0)
    sched = first + padm
    e_col = jax.lax.broadcasted_iota(jnp.int32, (E, 1), 0).astype(jnp.float32)
    order_col = jax.lax.dot_general(sched, e_col, (((1,), (0,)), ((), ())),
                                    preferred_element_type=jnp.float32)
    flag_col = (e_col >= npad).astype(jnp.float32)
    meta = jnp.concatenate([_t(order_col), _t(flag_col)], axis=0)
    meta_i = (meta + 0.5).astype(jnp.int32)
    coeff_se = jax.lax.dot_general(
        first, coeff, (((1,), (0,)), ((), ())),
        preferred_element_type=jnp.float32)
    return coeff_se, meta_i


def _shared_routing_kernel(x_ref, wg_ref, sw1_ref, sb1_ref, sw11_ref,
                           sb11_ref, sw2_ref, sb2_ref, z_ref, coeff_ref,
                           meta_ref):
    f = pl.program_id(0)
    xb = x_ref[...].astype(jnp.bfloat16)
    h1 = jax.lax.dot(xb, sw1_ref[...].astype(jnp.bfloat16),
                     preferred_element_type=jnp.float32) + sb1_ref[...]
    h2 = jax.lax.dot(xb, sw11_ref[...].astype(jnp.bfloat16),
                     preferred_element_type=jnp.float32) + sb11_ref[...]
    h = (h1 * jax.nn.sigmoid(h1)) * h2
    part = jax.lax.dot(h.astype(jnp.bfloat16), sw2_ref[...].astype(jnp.bfloat16),
                       preferred_element_type=jnp.float32)

    @pl.when(f == 0)
    def _first():
        z_ref[...] = sb2_ref[...] + part
        coeff_se, meta_i = _routing_body(x_ref[...], wg_ref[...])
        coeff_ref[...] = coeff_se[:, :, None]
        meta_ref[...] = meta_i

    @pl.when(f != 0)
    def _rest():
        z_ref[...] += part


def _moe_kernel(meta_ref, x_ref, w1_ref, b1_ref, w11_ref, b11_ref,
                w2_ref, b2_ref, coeff_ref, z_ref, y_ref):
    e = pl.program_id(0)

    @pl.when(e == 0)
    def _init():
        y_ref[...] = z_ref[...]

    @pl.when(meta_ref[1, e] == 1)
    def _compute():
        me = meta_ref[0, e]
        xb = x_ref[...].astype(jnp.bfloat16)
        h1 = jax.lax.dot(xb, w1_ref[0].astype(jnp.bfloat16),
                         preferred_element_type=jnp.float32) + b1_ref[me]
        h2 = jax.lax.dot(xb, w11_ref[0].astype(jnp.bfloat16),
                         preferred_element_type=jnp.float32) + b11_ref[me]
        h = (h1 * jax.nn.sigmoid(h1)) * h2
        part = jax.lax.dot(h.astype(jnp.bfloat16),
                           w2_ref[0].astype(jnp.bfloat16),
                           preferred_element_type=jnp.float32)
        part = part + b2_ref[me]
        y_ref[...] += coeff_ref[e] * part


def kernel(x, Wg, w1, b1, w11, b11, w2, b2, sw1, sb1, sw11, sb11, sw2, sb2):
    B, S, D = x.shape
    T = B * S
    E, _, F = w1.shape
    FS = sw1.shape[1]
    FT = F // _NF
    FTS = FS // _NFS
    xt = x.reshape(T, D)

    z, coeff, meta = pl.pallas_call(
        _shared_routing_kernel,
        grid=(_NFS,),
        in_specs=[
            pl.BlockSpec((T, D), lambda f: (0, 0)),
            pl.BlockSpec((D, E), lambda f: (0, 0)),
            pl.BlockSpec((D, FTS), lambda f: (0, f)),
            pl.BlockSpec((1, FTS), lambda f: (0, f)),
            pl.BlockSpec((D, FTS), lambda f: (0, f)),
            pl.BlockSpec((1, FTS), lambda f: (0, f)),
            pl.BlockSpec((FTS, D), lambda f: (f, 0)),
            pl.BlockSpec((1, D), lambda f: (0, 0)),
        ],
        out_specs=[
            pl.BlockSpec((T, D), lambda f: (0, 0)),
            pl.BlockSpec((E, T, 1), lambda f: (0, 0, 0)),
            pl.BlockSpec((2, E), lambda f: (0, 0)),
        ],
        out_shape=[
            jax.ShapeDtypeStruct((T, D), jnp.float32),
            jax.ShapeDtypeStruct((E, T, 1), jnp.float32),
            jax.ShapeDtypeStruct((2, E), jnp.int32),
        ],
        compiler_params=pltpu.CompilerParams(
            dimension_semantics=("arbitrary",)),
    )(xt, Wg, sw1, sb1.reshape(1, FS), sw11, sb11.reshape(1, FS), sw2,
      sb2.reshape(1, D))

    grid = (E,)

    def _wmap(e, m):
        return m[0, e], 0, 0

    def _w2map(e, m):
        return m[0, e], 0, 0

    y = pl.pallas_call(
        _moe_kernel,
        grid_spec=pltpu.PrefetchScalarGridSpec(
            num_scalar_prefetch=1,
            grid=grid,
            in_specs=[
                pl.BlockSpec((T, D), lambda e, m: (0, 0)),
                pl.BlockSpec((1, D, F), _wmap),
                pl.BlockSpec((E, 1, F), lambda e, m: (0, 0, 0)),
                pl.BlockSpec((1, D, F), _wmap),
                pl.BlockSpec((E, 1, F), lambda e, m: (0, 0, 0)),
                pl.BlockSpec((1, F, D), _w2map),
                pl.BlockSpec((E, 1, D), lambda e, m: (0, 0, 0)),
                pl.BlockSpec((E, T, 1), lambda e, m: (0, 0, 0)),
                pl.BlockSpec((T, D), lambda e, m: (0, 0)),
            ],
            out_specs=pl.BlockSpec((T, D), lambda e, m: (0, 0)),
        ),
        out_shape=jax.ShapeDtypeStruct((T, D), jnp.float32),
        compiler_params=pltpu.CompilerParams(
            dimension_semantics=("arbitrary",)),
    )(meta, xt, w1, b1.reshape(E, 1, F), w11, b11.reshape(E, 1, F),
      w2, b2.reshape(E, 1, D), coeff, z)

    return y.reshape(B, S, D)


# speedup vs baseline: 1.0271x; 1.0271x over previous
import jax
import jax.numpy as jnp
from jax.experimental import pallas as pl
from jax.experimental.pallas import tpu as pltpu

_E = 16
_NF = 1
_NFS = 2


def _routing_body(x, wg):
    T = x.shape[0]
    E = wg.shape[1]
    logits = jax.lax.dot_general(
        wg, x, (((0,), (1,)), ((), ())),
        preferred_element_type=jnp.float32)
    ids = jax.lax.broadcasted_iota(jnp.int32, (E, T), 0)
    m1 = jnp.max(logits, axis=0, keepdims=True)
    a = jnp.min(jnp.where(logits == m1, ids, E), axis=0, keepdims=True)
    masked = jnp.where(ids == a, -jnp.inf, logits)
    m2 = jnp.max(masked, axis=0, keepdims=True)
    b = jnp.min(jnp.where(masked == m2, ids, E), axis=0, keepdims=True)
    e2 = jnp.exp(m2 - m1)
    p1 = 1.0 / (1.0 + e2)
    p2 = e2 / (1.0 + e2)
    g0 = jnp.where(a == 0, p1, 0.0) + jnp.where(b == 0, p2, 0.0)
    g1 = jnp.where(a == 1, p1, 0.0) + jnp.where(b == 1, p2, 0.0)
    coeff = jnp.where(ids == a, g0, 0.0) + jnp.where(ids == b, g1, 0.0)

    act_col = (jnp.max(coeff, axis=1, keepdims=True) > 0.0).astype(jnp.float32)
    r = jax.lax.broadcasted_iota(jnp.int32, (E, E), 0)
    c = jax.lax.broadcasted_iota(jnp.int32, (E, E), 1)
    eye = (r == c).astype(jnp.float32)
    lstrict = (c < r).astype(jnp.float32)
    pos_col = jax.lax.dot_general(
        lstrict, act_col, (((1,), (0,)), ((), ())),
        preferred_element_type=jnp.float32)

    def _t(col):
        return jax.lax.dot_general(col, eye, (((0,), (0,)), ((), ())),
                                   preferred_element_type=jnp.float32)

    act_row = _t(act_col)
    pos_row = _t(pos_col)
    na = jnp.sum(act_row)
    rf = r.astype(jnp.float32)
    first = jnp.where((act_row > 0.0) & (pos_row == rf), 1.0, 0.0)
    padm = jnp.where((act_row > 0.0) & (pos_row == na - 1.0) & (rf >= na),
                     1.0, 0.0)
    sched = first + padm
    e_col = jax.lax.broadcasted_iota(jnp.int32, (E, 1), 0).astype(jnp.float32)
    order_col = jax.lax.dot_general(sched, e_col, (((1,), (0,)), ((), ())),
                                    preferred_element_type=jnp.float32)
    flag_col = (e_col < na).astype(jnp.float32)
    meta = jnp.concatenate([_t(order_col), _t(flag_col)], axis=0)
    meta_i = (meta + 0.5).astype(jnp.int32)
    coeff_se = jax.lax.dot_general(
        first, coeff, (((1,), (0,)), ((), ())),
        preferred_element_type=jnp.float32)
    return coeff_se, meta_i


def _shared_routing_kernel(x_ref, wg_ref, sw1_ref, sb1_ref, sw11_ref,
                           sb11_ref, sw2_ref, sb2_ref, z_ref, coeff_ref,
                           meta_ref):
    f = pl.program_id(0)
    xb = x_ref[...].astype(jnp.bfloat16)
    h1 = jax.lax.dot(xb, sw1_ref[...].astype(jnp.bfloat16),
                     preferred_element_type=jnp.float32) + sb1_ref[...]
    h2 = jax.lax.dot(xb, sw11_ref[...].astype(jnp.bfloat16),
                     preferred_element_type=jnp.float32) + sb11_ref[...]
    h = (h1 * jax.nn.sigmoid(h1)) * h2
    part = jax.lax.dot(h.astype(jnp.bfloat16), sw2_ref[...].astype(jnp.bfloat16),
                       preferred_element_type=jnp.float32)

    @pl.when(f == 0)
    def _first():
        z_ref[...] = sb2_ref[...] + part
        coeff_se, meta_i = _routing_body(x_ref[...], wg_ref[...])
        coeff_ref[...] = coeff_se[:, :, None]
        meta_ref[...] = meta_i

    @pl.when(f != 0)
    def _rest():
        z_ref[...] += part


def _moe_kernel(meta_ref, x_ref, w1_ref, b1_ref, w11_ref, b11_ref,
                w2_ref, b2_ref, coeff_ref, z_ref, y_ref):
    e = pl.program_id(0)

    @pl.when(e == 0)
    def _init():
        y_ref[...] = z_ref[...]

    @pl.when(meta_ref[1, e] == 1)
    def _compute():
        me = meta_ref[0, e]
        xb = x_ref[...].astype(jnp.bfloat16)
        h1 = jax.lax.dot(xb, w1_ref[0].astype(jnp.bfloat16),
                         preferred_element_type=jnp.float32) + b1_ref[me]
        h2 = jax.lax.dot(xb, w11_ref[0].astype(jnp.bfloat16),
                         preferred_element_type=jnp.float32) + b11_ref[me]
        h = (h1 * jax.nn.sigmoid(h1)) * h2
        part = jax.lax.dot(h.astype(jnp.bfloat16),
                           w2_ref[0].astype(jnp.bfloat16),
                           preferred_element_type=jnp.float32)
        part = part + b2_ref[me]
        y_ref[...] += coeff_ref[e] * part


def kernel(x, Wg, w1, b1, w11, b11, w2, b2, sw1, sb1, sw11, sb11, sw2, sb2):
    B, S, D = x.shape
    T = B * S
    E, _, F = w1.shape
    FS = sw1.shape[1]
    FT = F // _NF
    FTS = FS // _NFS
    xt = x.reshape(T, D)

    z, coeff, meta = pl.pallas_call(
        _shared_routing_kernel,
        grid=(_NFS,),
        in_specs=[
            pl.BlockSpec((T, D), lambda f: (0, 0)),
            pl.BlockSpec((D, E), lambda f: (0, 0)),
            pl.BlockSpec((D, FTS), lambda f: (0, f)),
            pl.BlockSpec((1, FTS), lambda f: (0, f)),
            pl.BlockSpec((D, FTS), lambda f: (0, f)),
            pl.BlockSpec((1, FTS), lambda f: (0, f)),
            pl.BlockSpec((FTS, D), lambda f: (f, 0)),
            pl.BlockSpec((1, D), lambda f: (0, 0)),
        ],
        out_specs=[
            pl.BlockSpec((T, D), lambda f: (0, 0)),
            pl.BlockSpec((E, T, 1), lambda f: (0, 0, 0)),
            pl.BlockSpec((2, E), lambda f: (0, 0)),
        ],
        out_shape=[
            jax.ShapeDtypeStruct((T, D), jnp.float32),
            jax.ShapeDtypeStruct((E, T, 1), jnp.float32),
            jax.ShapeDtypeStruct((2, E), jnp.int32),
        ],
        compiler_params=pltpu.CompilerParams(
            dimension_semantics=("arbitrary",)),
    )(xt, Wg, sw1, sb1.reshape(1, FS), sw11, sb11.reshape(1, FS), sw2,
      sb2.reshape(1, D))

    grid = (E,)

    def _wmap(e, m):
        return m[0, e], 0, 0

    def _w2map(e, m):
        return m[0, e], 0, 0

    y = pl.pallas_call(
        _moe_kernel,
        grid_spec=pltpu.PrefetchScalarGridSpec(
            num_scalar_prefetch=1,
            grid=grid,
            in_specs=[
                pl.BlockSpec((T, D), lambda e, m: (0, 0)),
                pl.BlockSpec((1, D, F), _wmap),
                pl.BlockSpec((E, 1, F), lambda e, m: (0, 0, 0)),
                pl.BlockSpec((1, D, F), _wmap),
                pl.BlockSpec((E, 1, F), lambda e, m: (0, 0, 0)),
                pl.BlockSpec((1, F, D), _w2map),
                pl.BlockSpec((E, 1, D), lambda e, m: (0, 0, 0)),
                pl.BlockSpec((E, T, 1), lambda e, m: (0, 0, 0)),
                pl.BlockSpec((T, D), lambda e, m: (0, 0)),
            ],
            out_specs=pl.BlockSpec((T, D), lambda e, m: (0, 0)),
        ),
        out_shape=jax.ShapeDtypeStruct((T, D), jnp.float32),
        compiler_params=pltpu.CompilerParams(
            dimension_semantics=("arbitrary",)),
    )(meta, xt, w1, b1.reshape(E, 1, F), w11, b11.reshape(E, 1, F),
      w2, b2.reshape(E, 1, D), coeff, z)

    return y.reshape(B, S, D)
